# baseline (speedup 1.0000x reference)
import functools

import jax
import jax.numpy as jnp
from jax import lax
from jax.experimental import pallas as pl
from jax.experimental.pallas import tpu as pltpu
from jax.experimental.pallas import tpu_sc as plsc

B = 32
S = 1
HIDDEN = 2048
DFF = 768
E = 64
TOPK = 8
T = B * S
L = 16
NV = E // L


def _logits_kernel(x_ref, gate_w_ref, out_ref):
    out_ref[...] = jax.lax.dot_general(
        x_ref[...], gate_w_ref[...],
        (((1,), (1,)), ((), ())),
        preferred_element_type=jnp.float32)


def _router_logits(x, gate_w):
    return pl.pallas_call(
        _logits_kernel,
        out_shape=jax.ShapeDtypeStruct((T, E), jnp.float32),
    )(x, gate_w)


def _sc_topk_body(logits_hbm, out_hbm, row_v, w_v):
    t = lax.axis_index("s") * 2 + lax.axis_index("c")
    pltpu.sync_copy(logits_hbm.at[t], row_v)
    rows = [row_v[pl.ds(i * L, L)] for i in range(NV)]
    fiotas = [lax.iota(jnp.int32, L).astype(jnp.float32) + jnp.float32(i * L)
              for i in range(NV)]
    neg_inf = jnp.float32(-jnp.inf)
    big = jnp.float32(E)
    _dnums = lax.GatherDimensionNumbers(
        offset_dims=(), collapsed_slice_dims=(0,), start_index_map=(0,))
    iota16 = lax.iota(jnp.int32, L)

    def _shuf(v, s):
        idx = (iota16 ^ s).reshape(L, 1)
        return lax.gather(v, idx, _dnums, (1,),
                          mode=lax.GatherScatterMode.PROMISE_IN_BOUNDS)

    def _allreduce(v, op):
        for s in (1, 2, 4, 8):
            v = op(v, _shuf(v, s))
        return v

    def bcast_max(v):
        return _allreduce(v, jnp.maximum)

    def bcast_min(v):
        return _allreduce(v, jnp.minimum)

    def bcast_sum(v):
        return _allreduce(v, jnp.add)

    cur = list(rows)
    for _ in range(TOPK):
        mx = bcast_max(jnp.maximum(jnp.maximum(cur[0], cur[1]),
                                   jnp.maximum(cur[2], cur[3])))
        cands = [jnp.where(cur[i] == mx, fiotas[i], big) for i in range(NV)]
        pos = bcast_min(jnp.minimum(jnp.minimum(cands[0], cands[1]),
                                    jnp.minimum(cands[2], cands[3])))
        for i in range(NV):
            cur[i] = jnp.where(fiotas[i] == pos, neg_inf, cur[i])
    z = [jnp.where(cur[i] == neg_inf, rows[i], neg_inf) for i in range(NV)]
    zmax = bcast_max(jnp.maximum(jnp.maximum(z[0], z[1]),
                                 jnp.maximum(z[2], z[3])))
    p = [jnp.where(cur[i] == neg_inf, jnp.exp(z[i] - zmax), jnp.float32(0.0))
         for i in range(NV)]
    denom = bcast_sum(p[0] + p[1] + p[2] + p[3])
    for i in range(NV):
        w_v[pl.ds(i * L, L)] = p[i] / denom
    pltpu.sync_copy(w_v, out_hbm.at[t])


def _sc_router_weights(logits):
    mesh = plsc.VectorSubcoreMesh(core_axis_name="c", subcore_axis_name="s")
    return pl.kernel(
        _sc_topk_body,
        mesh=mesh,
        out_type=jax.ShapeDtypeStruct((T, E), jnp.float32),
        scratch_types=[
            pltpu.VMEM((E,), jnp.float32),
            pltpu.VMEM((E,), jnp.float32),
        ],
    )(logits)


def _moe_kernel(x_ref, rw_ref, wg_ref, wu_ref, wd_ref, out_ref, acc_ref):
    e = pl.program_id(0)

    @pl.when(e == 0)
    def _init():
        acc_ref[...] = jnp.zeros_like(acc_ref)

    x = x_ref[...]
    g = jax.lax.dot_general(x, wg_ref[0], (((1,), (1,)), ((), ())),
                            preferred_element_type=jnp.float32)
    u = jax.lax.dot_general(x, wu_ref[0], (((1,), (1,)), ((), ())),
                            preferred_element_type=jnp.float32)
    glu = g * jax.nn.sigmoid(g) * u
    o = jax.lax.dot_general(glu, wd_ref[0], (((1,), (1,)), ((), ())),
                            preferred_element_type=jnp.float32)
    rw = rw_ref[...]
    ecol = jax.lax.broadcasted_iota(jnp.int32, (T, E), 1)
    w_col = jnp.sum(jnp.where(ecol == e, rw, 0.0), axis=1, keepdims=True)
    acc_ref[...] += w_col * o

    @pl.when(e == E - 1)
    def _write():
        out_ref[...] = acc_ref[...]


def kernel(hidden_states, gate_w, w_gate, w_up, w_down):
    x = hidden_states.reshape(T, HIDDEN)
    logits = _router_logits(x, gate_w)
    rw = _sc_router_weights(logits)
    out = pl.pallas_call(
        _moe_kernel,
        grid=(E,),
        in_specs=[
            pl.BlockSpec((T, HIDDEN), lambda e: (0, 0)),
            pl.BlockSpec((T, E), lambda e: (0, 0)),
            pl.BlockSpec((1, DFF, HIDDEN), lambda e: (e, 0, 0)),
            pl.BlockSpec((1, DFF, HIDDEN), lambda e: (e, 0, 0)),
            pl.BlockSpec((1, HIDDEN, DFF), lambda e: (e, 0, 0)),
        ],
        out_specs=pl.BlockSpec((T, HIDDEN), lambda e: (0, 0)),
        out_shape=jax.ShapeDtypeStruct((T, HIDDEN), jnp.float32),
        scratch_shapes=[
            pltpu.VMEM((T, HIDDEN), jnp.float32),
        ],
    )(x, rw, w_gate, w_up, w_down)
    return out.reshape(B, S, HIDDEN)


# speedup vs baseline: 1.0480x; 1.0480x over previous
import jax
import jax.numpy as jnp
from jax.experimental import pallas as pl
from jax.experimental.pallas import tpu as pltpu

B = 32
S = 1
HIDDEN = 2048
DFF = 768
E = 64
TOPK = 8
T = B * S

_DOT_PREC = jax.lax.Precision.DEFAULT


def _moe_kernel(x_ref, gate_w_ref, wg_ref, wu_ref, wd_ref, out_ref,
                rw_ref, acc_ref):
    e = pl.program_id(0)

    @pl.when(e == 0)
    def _router():
        x = x_ref[...]
        logits = jax.lax.dot_general(
            x, gate_w_ref[...],
            (((1,), (1,)), ((), ())),
            preferred_element_type=jnp.float32)
        col = jax.lax.broadcasted_iota(jnp.int32, (T, E), 1)
        neg_inf = jnp.float32(-jnp.inf)
        cur = logits
        sel = jnp.zeros((T, E), dtype=jnp.bool_)
        for _ in range(TOPK):
            mx = jnp.max(cur, axis=1, keepdims=True)
            at_max = cur == mx
            first = jnp.min(jnp.where(at_max, col, E), axis=1, keepdims=True)
            pick = col == first
            sel = jnp.logical_or(sel, pick)
            cur = jnp.where(pick, neg_inf, cur)
        z = jnp.where(sel, logits, neg_inf)
        zmax = jnp.max(z, axis=1, keepdims=True)
        p = jnp.where(sel, jnp.exp(z - zmax), 0.0)
        rw_ref[...] = p / jnp.sum(p, axis=1, keepdims=True)
        acc_ref[...] = jnp.zeros_like(acc_ref)

    x = x_ref[...]
    g = jax.lax.dot_general(x, wg_ref[0], (((1,), (1,)), ((), ())),
                            preferred_element_type=jnp.float32,
                            precision=_DOT_PREC)
    u = jax.lax.dot_general(x, wu_ref[0], (((1,), (1,)), ((), ())),
                            preferred_element_type=jnp.float32,
                            precision=_DOT_PREC)
    rw = rw_ref[...]
    ecol = jax.lax.broadcasted_iota(jnp.int32, (T, E), 1)
    w_col = jnp.sum(jnp.where(ecol == e, rw, 0.0), axis=1, keepdims=True)
    glu = w_col * (g * jax.nn.sigmoid(g) * u)
    o = jax.lax.dot_general(glu, wd_ref[0], (((1,), (1,)), ((), ())),
                            preferred_element_type=jnp.float32,
                            precision=_DOT_PREC)
    acc_ref[...] += o

    @pl.when(e == E - 1)
    def _write():
        out_ref[...] = acc_ref[...]


def kernel(hidden_states, gate_w, w_gate, w_up, w_down):
    x = hidden_states.reshape(T, HIDDEN)
    out = pl.pallas_call(
        _moe_kernel,
        grid=(E,),
        in_specs=[
            pl.BlockSpec((T, HIDDEN), lambda e: (0, 0)),
            pl.BlockSpec((E, HIDDEN), lambda e: (0, 0)),
            pl.BlockSpec((1, DFF, HIDDEN), lambda e: (e, 0, 0)),
            pl.BlockSpec((1, DFF, HIDDEN), lambda e: (e, 0, 0)),
            pl.BlockSpec((1, HIDDEN, DFF), lambda e: (e, 0, 0)),
        ],
        out_specs=pl.BlockSpec((T, HIDDEN), lambda e: (0, 0)),
        out_shape=jax.ShapeDtypeStruct((T, HIDDEN), jnp.float32),
        scratch_shapes=[
            pltpu.VMEM((T, E), jnp.float32),
            pltpu.VMEM((T, HIDDEN), jnp.float32),
        ],
    )(x, gate_w, w_gate, w_up, w_down)
    return out.reshape(B, S, HIDDEN)


# speedup vs baseline: 1.0483x; 1.0003x over previous
import jax
import jax.numpy as jnp
from jax.experimental import pallas as pl
from jax.experimental.pallas import tpu as pltpu

B = 32
S = 1
HIDDEN = 2048
DFF = 768
E = 64
TOPK = 8
T = B * S

_DOT_PREC = jax.lax.Precision.DEFAULT


def _moe_kernel(x_ref, gate_w_ref, wg_ref, wu_ref, wd_ref, out_ref,
                rw_ref, acc_ref):
    e = pl.program_id(0)

    @pl.when(e == 0)
    def _router():
        x = x_ref[...]
        logits = jax.lax.dot_general(
            x, gate_w_ref[...],
            (((1,), (1,)), ((), ())),
            preferred_element_type=jnp.float32)
        col = jax.lax.broadcasted_iota(jnp.int32, (T, E), 1)
        neg_inf = jnp.float32(-jnp.inf)
        cur = logits
        sel = jnp.zeros((T, E), dtype=jnp.bool_)
        for _ in range(TOPK):
            mx = jnp.max(cur, axis=1, keepdims=True)
            at_max = cur == mx
            first = jnp.min(jnp.where(at_max, col, E), axis=1, keepdims=True)
            pick = col == first
            sel = jnp.logical_or(sel, pick)
            cur = jnp.where(pick, neg_inf, cur)
        z = jnp.where(sel, logits, neg_inf)
        zmax = jnp.max(z, axis=1, keepdims=True)
        p = jnp.where(sel, jnp.exp(z - zmax), 0.0)
        rw_ref[...] = p / jnp.sum(p, axis=1, keepdims=True)
        acc_ref[...] = jnp.zeros_like(acc_ref)

    x = x_ref[...]
    g = jax.lax.dot_general(x, wg_ref[0], (((1,), (1,)), ((), ())),
                            preferred_element_type=jnp.float32,
                            precision=_DOT_PREC)
    u = jax.lax.dot_general(x, wu_ref[0], (((1,), (1,)), ((), ())),
                            preferred_element_type=jnp.float32,
                            precision=_DOT_PREC)
    glu = g * jax.nn.sigmoid(g) * u
    o = jax.lax.dot_general(glu, wd_ref[0], (((1,), (1,)), ((), ())),
                            preferred_element_type=jnp.float32,
                            precision=_DOT_PREC)
    rw = rw_ref[...]
    ecol = jax.lax.broadcasted_iota(jnp.int32, (T, E), 1)
    w_col = jnp.sum(jnp.where(ecol == e, rw, 0.0), axis=1, keepdims=True)
    acc_ref[...] += w_col * o

    @pl.when(e == E - 1)
    def _write():
        out_ref[...] = acc_ref[...]


def kernel(hidden_states, gate_w, w_gate, w_up, w_down):
    x = hidden_states.reshape(T, HIDDEN)
    out = pl.pallas_call(
        _moe_kernel,
        grid=(E,),
        in_specs=[
            pl.BlockSpec((T, HIDDEN), lambda e: (0, 0)),
            pl.BlockSpec((E, HIDDEN), lambda e: (0, 0)),
            pl.BlockSpec((1, DFF, HIDDEN), lambda e: (e, 0, 0)),
            pl.BlockSpec((1, DFF, HIDDEN), lambda e: (e, 0, 0)),
            pl.BlockSpec((1, HIDDEN, DFF), lambda e: (e, 0, 0)),
        ],
        out_specs=pl.BlockSpec((T, HIDDEN), lambda e: (0, 0)),
        out_shape=jax.ShapeDtypeStruct((T, HIDDEN), jnp.float32),
        scratch_shapes=[
            pltpu.VMEM((T, E), jnp.float32),
            pltpu.VMEM((T, HIDDEN), jnp.float32),
        ],
    )(x, gate_w, w_gate, w_up, w_down)
    return out.reshape(B, S, HIDDEN)
